# baseline (speedup 1.0000x reference)
import functools

import jax
import jax.numpy as jnp
from jax import lax
from jax.experimental import pallas as pl
from jax.experimental.pallas import tpu as pltpu
from jax.experimental.pallas import tpu_sc as plsc

N = 10000
NP = 10240
E = 320000
EP = 327680
G = 64
D = 128
NC = 2
NS = 16
NW = NC * NS
BPW = NP // NW
EPW = EP // NW
EC = 128
NCHUNK = EPW // EC
RPT = NP // NS
PAD_NODE = 10200

BK1 = 1024
BK = 512
NB = NP // BK
JC = 1024
NEG = -2.0
DT = 144

@functools.lru_cache(maxsize=None)
def _sc_kernels():
    mesh = plsc.VectorSubcoreMesh(core_axis_name="c", subcore_axis_name="s",
                                  num_cores=NC, num_subcores=NS)

    @functools.partial(
        pl.kernel,
        out_type=jax.ShapeDtypeStruct((NP, D), jnp.float32),
        mesh=mesh,
        scratch_types=[
            pltpu.VMEM((BPW,), jnp.int32),
            pltpu.VMEM((BPW, D), jnp.float32),
            pltpu.SemaphoreType.DMA,
        ],
    )
    def _emb_gather(table_hbm, idx_hbm, out_hbm, idx_v, rows_v, sem):
        wid = lax.axis_index("s") * NC + lax.axis_index("c")
        base = wid * BPW
        pltpu.sync_copy(idx_hbm.at[pl.ds(base, BPW)], idx_v)
        for off, sz in ((0, 128), (128, 128), (256, 64)):
            pltpu.async_copy(
                table_hbm.at[idx_v.at[pl.ds(off, sz)]],
                rows_v.at[pl.ds(off, sz)],
                sem,
            ).wait()
        pltpu.sync_copy(rows_v, out_hbm.at[pl.ds(base, BPW)])

    @functools.partial(
        pl.kernel,
        out_type=[
            jax.ShapeDtypeStruct((NC, NP, DT), jnp.float32),
        ],
        mesh=mesh,
        scratch_types=(
            [pltpu.VMEM((EC, DT), jnp.float32)] * 2
            + [pltpu.VMEM((EC,), jnp.int32)] * 8
            + [pltpu.VMEM_SHARED((NP, DT), jnp.float32)]
            + [pltpu.SemaphoreType.DMA] * 6
        ),
        compiler_params=pltpu.CompilerParams(use_tc_tiling_on_sc=False),
    )
    def _edge_agg(h_hbm, src_hbm, dst_hbm, agg_out,
                  rows_0, rows_1,
                  sidx_0, sidx_1, sidx_2, sidx_3,
                  didx_0, didx_1, didx_2, didx_3,
                  acc_sh,
                  sem_gr0, sem_gr1,
                  sem_i0, sem_i1, sem_i2, sem_i3):
        rows = (rows_0, rows_1)
        sidx = (sidx_0, sidx_1, sidx_2, sidx_3)
        didx = (didx_0, didx_1, didx_2, didx_3)
        sem_gr = (sem_gr0, sem_gr1)
        sem_i = (sem_i0, sem_i1, sem_i2, sem_i3)

        core = lax.axis_index("c")
        sub = lax.axis_index("s")
        wid = sub * NC + core
        ebase = wid * EPW

        zv = jnp.zeros((16,), jnp.float32)

        def zrow(i, c):
            rows_0[i // 9, pl.ds((i % 9) * 16, 16)] = zv
            return c

        lax.fori_loop(0, EC * 9, zrow, 0)

        for r in range(RPT // EC):
            pltpu.sync_copy(rows_0,
                            acc_sh.at[pl.ds(sub * RPT + r * EC, EC)])
        plsc.subcore_barrier()

        def issue_idx(c, q):
            pltpu.async_copy(src_hbm.at[pl.ds(ebase + c * EC, EC)],
                             sidx[q], sem_i[q])
            pltpu.async_copy(dst_hbm.at[pl.ds(ebase + c * EC, EC)],
                             didx[q], sem_i[q])

        def wait_idx(q):
            pltpu.make_async_copy(src_hbm.at[pl.ds(0, EC)], sidx[q],
                                  sem_i[q]).wait()
            pltpu.make_async_copy(dst_hbm.at[pl.ds(0, EC)], didx[q],
                                  sem_i[q]).wait()

        def issue_g(q, b):
            pltpu.async_copy(h_hbm.at[sidx[q]], rows[b], sem_gr[b])

        def wait_g(b):
            pltpu.make_async_copy(h_hbm.at[sidx[0]], rows[b],
                                  sem_gr[b]).wait()

        def sync_s(q, b):
            pltpu.sync_copy(rows[b], acc_sh.at[didx[q]], add=True)

        for q in range(4):
            issue_idx(q, q)
        wait_idx(0)
        issue_g(0, 0)
        wait_idx(1)
        issue_g(1, 1)

        def ring(g, carry):
            for bb in range(4):
                c = g * 4 + bb
                b = bb % 2
                wait_g(b)
                sync_s(bb, b)

                @pl.when(c + 2 < NCHUNK)
                def _(c=c, bb=bb, b=b):
                    wait_idx((bb + 2) % 4)
                    issue_g((bb + 2) % 4, b)

                @pl.when(c + 4 < NCHUNK)
                def _(c=c, bb=bb):
                    issue_idx(c + 4, bb)
            return carry

        lax.fori_loop(0, NCHUNK // 4, ring, 0)

        plsc.subcore_barrier()
        pltpu.sync_copy(acc_sh.at[pl.ds(sub * RPT, RPT)],
                        agg_out.at[core, pl.ds(sub * RPT, RPT)])

    return _emb_gather, _edge_agg


def _k1_body(aggP, h, alive, Wl, bl, Wr, p, hout_ref, score_ref):
    aggc = aggP[0] + aggP[1]
    agg = aggc[:, 0:D]
    cnt = aggc[:, D:D + 1]
    mean = agg / jnp.maximum(cnt, 1.0)
    pre = mean @ Wl[...].T + bl[...] + h[...][:, 0:D] @ Wr[...].T
    ho = jnp.maximum(pre, 0.0)
    hout_ref[...] = ho
    pv = p[...]
    pn = jnp.sqrt(jnp.sum(pv * pv))
    s = jnp.tanh(jnp.sum(ho * pv, axis=1, keepdims=True) / pn)
    score_ref[...] = jnp.where(alive[...] > 0.0, s, NEG)


_k1 = pl.pallas_call(
    _k1_body,
    grid=(NP // BK1,),
    in_specs=[
        pl.BlockSpec((NC, BK1, DT), lambda i: (0, i, 0)),
        pl.BlockSpec((BK1, DT), lambda i: (i, 0)),
        pl.BlockSpec((BK1, 1), lambda i: (i, 0)),
        pl.BlockSpec((D, D), lambda i: (0, 0)),
        pl.BlockSpec((1, D), lambda i: (0, 0)),
        pl.BlockSpec((D, D), lambda i: (0, 0)),
        pl.BlockSpec((1, D), lambda i: (0, 0)),
    ],
    out_specs=[
        pl.BlockSpec((BK1, D), lambda i: (i, 0)),
        pl.BlockSpec((BK1, 1), lambda i: (i, 0)),
    ],
    out_shape=[
        jax.ShapeDtypeStruct((NP, D), jnp.float32),
        jax.ShapeDtypeStruct((NP, 1), jnp.float32),
    ],
)


def _k23_body(s_col, s_row, b_col, b_row, p_col, p_row, hout,
              hp_out, keep_out, prio_out, xs_out,
              kr_s, Kr_s, kc_s, st_s, cf_s, S_s, M_s):
    step = pl.program_id(0)

    @pl.when(step == 0)
    def _():
        sr = s_row[...]
        br = b_row[...]
        gi_col = lax.broadcasted_iota(jnp.int32, (G, 1), 0).astype(jnp.float32)
        alive_r = jnp.where(sr > -1.5, 1.0, 0.0)
        oh_gn = jnp.where(br == gi_col, 1.0, 0.0)
        i0 = lax.broadcasted_iota(jnp.int32, (G, G), 0)
        i1 = lax.broadcasted_iota(jnp.int32, (G, G), 1)
        eye = jnp.where(i0 == i1, 1.0, 0.0)
        ut = jnp.where(i0 < i1, 1.0, 0.0)
        ones_g = jnp.ones((1, G), jnp.float32)

        def tr(col):
            return ones_g @ (eye * col)

        kc = jnp.ceil(jnp.float32(0.8)
                      * jnp.sum(oh_gn * alive_r, axis=1, keepdims=True))
        kc_s[...] = kc
        kr = tr(kc)
        kr_s[...] = kr
        Kr_s[...] = kr @ ut
        cf = tr(jnp.sum(oh_gn, axis=1, keepdims=True))
        cf_s[...] = cf
        st_s[...] = cf @ ut
        S_s[...] = jnp.zeros((G, D), jnp.float32)
        M_s[...] = jnp.full((G, D), -jnp.inf, jnp.float32)

    @pl.when(step > 0)
    def _():
        i = step - 1
        si = s_col[...]
        bi = b_col[...]
        pi = p_col[...]
        gi_row = lax.broadcasted_iota(jnp.int32, (1, G), 1).astype(jnp.float32)
        gi_col = lax.broadcasted_iota(jnp.int32, (G, 1), 0).astype(jnp.float32)
        bmin = bi[0, 0]
        bmax = bi[BK - 1, 0]
        lo = jnp.sum(jnp.where(gi_row == bmin, st_s[...], 0.0))
        hi = jnp.sum(jnp.where(gi_row == bmax, st_s[...] + cf_s[...], 0.0))
        jlo = lo.astype(jnp.int32) // JC
        jhi = (hi.astype(jnp.int32) + JC - 1) // JC

        def jbody(jc, rank):
            sj = s_row[:, pl.ds(jc * JC, JC)]
            bj = b_row[:, pl.ds(jc * JC, JC)]
            pj = p_row[:, pl.ds(jc * JC, JC)]
            cmp = (bj == bi) & ((sj > si) | ((sj == si) & (pj < pi)))
            return rank + jnp.sum(jnp.where(cmp, 1.0, 0.0), axis=1,
                                  keepdims=True)

        rank = lax.fori_loop(jlo, jhi, jbody, jnp.zeros((BK, 1), jnp.float32))

        ohi = jnp.where(bi == gi_row, 1.0, 0.0)
        k_i = jnp.sum(ohi * kr_s[...], axis=1, keepdims=True)
        K_i = jnp.sum(ohi * Kr_s[...], axis=1, keepdims=True)
        keep = rank < k_i
        keepf = jnp.where(keep, 1.0, 0.0)
        gate = jnp.where(keep, si, 0.0)
        hp = hout[...] * gate
        hp_out[:, 0:D] = hp
        hp_out[:, D:D + 1] = keepf
        hp_out[:, D + 1:DT] = jnp.zeros((BK, DT - D - 1), jnp.float32)
        keep_out[...] = keepf
        prio_out[...] = K_i + rank

        brow_i = b_row[:, pl.ds(i * BK, BK)]
        ohg = jnp.where(brow_i == gi_col, 1.0, 0.0)
        S_s[...] = S_s[...] + ohg @ hp

        for g in range(G):
            @pl.when((bmin <= jnp.float32(g)) & (jnp.float32(g) <= bmax))
            def _(g=g):
                msk = (bi == jnp.float32(g)) & keep
                mrow = jnp.max(jnp.where(msk, hp, -jnp.inf), axis=0,
                               keepdims=True)
                M_s[g:g + 1, :] = jnp.maximum(M_s[g:g + 1, :], mrow)

    @pl.when(step == NB)
    def _():
        mean = S_s[...] / jnp.maximum(kc_s[...], 1.0)
        mx = M_s[...]
        mx = jnp.where(mx == -jnp.inf, 0.0, mx)
        xs_out[:, 0:D] = mx
        xs_out[:, D:2 * D] = mean


def _bk_map(i):
    return (jnp.maximum(i - 1, 0), 0)


_k23 = pl.pallas_call(
    _k23_body,
    grid=(NB + 1,),
    in_specs=[
        pl.BlockSpec((BK, 1), _bk_map),
        pl.BlockSpec((1, NP), lambda i: (0, 0)),
        pl.BlockSpec((BK, 1), _bk_map),
        pl.BlockSpec((1, NP), lambda i: (0, 0)),
        pl.BlockSpec((BK, 1), _bk_map),
        pl.BlockSpec((1, NP), lambda i: (0, 0)),
        pl.BlockSpec((BK, D), _bk_map),
    ],
    out_specs=[
        pl.BlockSpec((BK, DT), _bk_map),
        pl.BlockSpec((BK, 1), _bk_map),
        pl.BlockSpec((BK, 1), _bk_map),
        pl.BlockSpec((G, 2 * D), lambda i: (0, 0)),
    ],
    out_shape=[
        jax.ShapeDtypeStruct((NP, DT), jnp.float32),
        jax.ShapeDtypeStruct((NP, 1), jnp.float32),
        jax.ShapeDtypeStruct((NP, 1), jnp.float32),
        jax.ShapeDtypeStruct((G, 2 * D), jnp.float32),
    ],
    scratch_shapes=[
        pltpu.VMEM((1, G), jnp.float32),
        pltpu.VMEM((1, G), jnp.float32),
        pltpu.VMEM((G, 1), jnp.float32),
        pltpu.VMEM((1, G), jnp.float32),
        pltpu.VMEM((1, G), jnp.float32),
        pltpu.VMEM((G, D), jnp.float32),
        pltpu.VMEM((G, D), jnp.float32),
    ],
)


def _k4_body(z1, z2, z3, W1, b1, W2, b2, W3, b3, o_ref):
    z = z1[...] + z2[...] + z3[...]
    h = jnp.maximum(z @ W1[...].T + b1[...], 0.0)
    h = jnp.maximum(h @ W2[...].T + b2[...], 0.0)
    s = jnp.sum(h * W3[...], axis=1, keepdims=True) + b3[...]
    o_ref[...] = jax.nn.sigmoid(s)


_k4 = pl.pallas_call(
    _k4_body,
    out_shape=jax.ShapeDtypeStruct((G, 1), jnp.float32),
)


def kernel(x, edge_index, batch, emb, Wl1, bl1, Wr1, p1, Wl2, bl2, Wr2, p2,
           Wl3, bl3, Wr3, p3, W1, b1, W2, b2, W3, b3):
    emb_gather, edge_agg = _sc_kernels()
    ids_p = jnp.concatenate(
        [x[:, 0].astype(jnp.int32), jnp.zeros((NP - N,), jnp.int32)])
    h0 = emb_gather(emb, ids_p)

    epad = jnp.full((EP - E,), PAD_NODE, jnp.int32)
    src_p = jnp.concatenate([edge_index[0].astype(jnp.int32), epad])
    dst_p = jnp.concatenate([edge_index[1].astype(jnp.int32), epad])

    batch_pf = jnp.concatenate(
        [batch.astype(jnp.float32), jnp.full((NP - N,), 63.0, jnp.float32)])
    b_col = batch_pf.reshape(NP, 1)
    b_row = batch_pf.reshape(1, NP)
    prio = jnp.arange(NP, dtype=jnp.float32)
    p_col = prio.reshape(NP, 1)
    p_row = prio.reshape(1, NP)
    alive_col = (jnp.arange(NP) < N).astype(jnp.float32).reshape(NP, 1)
    h = jnp.concatenate(
        [h0, alive_col, jnp.zeros((NP, DT - D - 1), jnp.float32)], axis=1)

    xs_list = []
    for (Wl, bl, Wr, p) in ((Wl1, bl1, Wr1, p1), (Wl2, bl2, Wr2, p2),
                            (Wl3, bl3, Wr3, p3)):
        (aggP,) = edge_agg(h, src_p, dst_p)
        hout, s_col = _k1(aggP, h, alive_col,
                          Wl, bl.reshape(1, D), Wr, p.reshape(1, D))
        s_row = s_col.reshape(1, NP)
        h, keep_col, prio_col, xs = _k23(s_col, s_row, b_col, b_row,
                                         p_col, p_row, hout)
        xs_list.append(xs)
        alive_col = keep_col
        p_col = prio_col
        p_row = prio_col.reshape(1, NP)

    out = _k4(xs_list[0], xs_list[1], xs_list[2],
              W1, b1.reshape(1, -1), W2, b2.reshape(1, -1),
              W3, b3.reshape(1, 1))
    return out[:, 0]


# speedup vs baseline: 1.1461x; 1.1461x over previous
import functools

import jax
import jax.numpy as jnp
from jax import lax
from jax.experimental import pallas as pl
from jax.experimental.pallas import tpu as pltpu
from jax.experimental.pallas import tpu_sc as plsc

N = 10000
NP = 10240
E = 320000
EP = 327680
G = 64
D = 128
NC = 2
NS = 16
NW = NC * NS
BPW = NP // NW
EPW = EP // NW
EC = 128
NCHUNK = EPW // EC
RPT = NP // NS
PAD_NODE = 10200

BK1 = 1024
BK = 512
NB = NP // BK
JC = 1024
NEG = -2.0
DT = 144

@functools.lru_cache(maxsize=None)
def _sc_kernels():
    mesh = plsc.VectorSubcoreMesh(core_axis_name="c", subcore_axis_name="s",
                                  num_cores=NC, num_subcores=NS)

    @functools.partial(
        pl.kernel,
        out_type=jax.ShapeDtypeStruct((NP, D), jnp.float32),
        mesh=mesh,
        scratch_types=[
            pltpu.VMEM((BPW,), jnp.int32),
            pltpu.VMEM((BPW, D), jnp.float32),
            pltpu.SemaphoreType.DMA,
        ],
    )
    def _emb_gather(table_hbm, idx_hbm, out_hbm, idx_v, rows_v, sem):
        wid = lax.axis_index("s") * NC + lax.axis_index("c")
        base = wid * BPW
        pltpu.sync_copy(idx_hbm.at[pl.ds(base, BPW)], idx_v)
        for off, sz in ((0, 128), (128, 128), (256, 64)):
            pltpu.async_copy(
                table_hbm.at[idx_v.at[pl.ds(off, sz)]],
                rows_v.at[pl.ds(off, sz)],
                sem,
            ).wait()
        pltpu.sync_copy(rows_v, out_hbm.at[pl.ds(base, BPW)])

    @functools.partial(
        pl.kernel,
        out_type=[
            jax.ShapeDtypeStruct((NC, NP, D), jnp.float32),
            jax.ShapeDtypeStruct((NC, NP, 16), jnp.float32),
        ],
        mesh=mesh,
        scratch_types=(
            [pltpu.VMEM((EC, D), jnp.float32)] * 2
            + [pltpu.VMEM((EC, 16), jnp.float32)] * 2
            + [pltpu.VMEM((EC,), jnp.int32)] * 8
            + [
                pltpu.VMEM_SHARED((NP, D), jnp.float32),
                pltpu.VMEM_SHARED((NP, 16), jnp.float32),
            ]
            + [pltpu.SemaphoreType.DMA] * 8
        ),
        compiler_params=pltpu.CompilerParams(use_tc_tiling_on_sc=False),
    )
    def _edge_agg(h_hbm, alive16_hbm, src_hbm, dst_hbm, agg_out, cnt_out,
                  rows_0, rows_1, arows_0, arows_1,
                  sidx_0, sidx_1, sidx_2, sidx_3,
                  didx_0, didx_1, didx_2, didx_3,
                  acc_sh, cnt_sh,
                  sem_gr0, sem_gr1, sem_ga0, sem_ga1,
                  sem_i0, sem_i1, sem_i2, sem_i3):
        rows = (rows_0, rows_1)
        arows = (arows_0, arows_1)
        sidx = (sidx_0, sidx_1, sidx_2, sidx_3)
        didx = (didx_0, didx_1, didx_2, didx_3)
        sem_gr = (sem_gr0, sem_gr1)
        sem_ga = (sem_ga0, sem_ga1)
        sem_i = (sem_i0, sem_i1, sem_i2, sem_i3)

        core = lax.axis_index("c")
        sub = lax.axis_index("s")
        wid = sub * NC + core
        ebase = wid * EPW

        zv = jnp.zeros((16,), jnp.float32)

        def zrow(i, c):
            rows_0[i // 8, pl.ds((i % 8) * 16, 16)] = zv
            return c

        lax.fori_loop(0, EC * 8, zrow, 0)

        def zarow(i, c):
            arows_0[i, pl.ds(0, 16)] = zv
            return c

        lax.fori_loop(0, EC, zarow, 0)

        for r in range(RPT // EC):
            pltpu.sync_copy(rows_0,
                            acc_sh.at[pl.ds(sub * RPT + r * EC, EC)])
            pltpu.sync_copy(arows_0,
                            cnt_sh.at[pl.ds(sub * RPT + r * EC, EC)])
        plsc.subcore_barrier()

        def issue_idx(c, q):
            pltpu.async_copy(src_hbm.at[pl.ds(ebase + c * EC, EC)],
                             sidx[q], sem_i[q])
            pltpu.async_copy(dst_hbm.at[pl.ds(ebase + c * EC, EC)],
                             didx[q], sem_i[q])

        def wait_idx(q):
            pltpu.make_async_copy(src_hbm.at[pl.ds(0, EC)], sidx[q],
                                  sem_i[q]).wait()
            pltpu.make_async_copy(dst_hbm.at[pl.ds(0, EC)], didx[q],
                                  sem_i[q]).wait()

        def issue_g(q, b):
            pltpu.async_copy(h_hbm.at[sidx[q]], rows[b], sem_gr[b])
            pltpu.async_copy(alive16_hbm.at[sidx[q]], arows[b], sem_ga[b])

        def wait_g(b):
            pltpu.make_async_copy(h_hbm.at[sidx[0]], rows[b],
                                  sem_gr[b]).wait()
            pltpu.make_async_copy(alive16_hbm.at[sidx[0]], arows[b],
                                  sem_ga[b]).wait()

        def sync_s(q, b):
            pltpu.sync_copy(rows[b], acc_sh.at[didx[q]], add=True)
            pltpu.sync_copy(arows[b], cnt_sh.at[didx[q]], add=True)

        for q in range(4):
            issue_idx(q, q)
        wait_idx(0)
        issue_g(0, 0)
        wait_idx(1)
        issue_g(1, 1)

        def ring(g, carry):
            for bb in range(4):
                c = g * 4 + bb
                b = bb % 2
                wait_g(b)
                sync_s(bb, b)

                @pl.when(c + 2 < NCHUNK)
                def _(c=c, bb=bb, b=b):
                    wait_idx((bb + 2) % 4)
                    issue_g((bb + 2) % 4, b)

                @pl.when(c + 4 < NCHUNK)
                def _(c=c, bb=bb):
                    issue_idx(c + 4, bb)
            return carry

        lax.fori_loop(0, NCHUNK // 4, ring, 0)

        plsc.subcore_barrier()
        pltpu.sync_copy(acc_sh.at[pl.ds(sub * RPT, RPT)],
                        agg_out.at[core, pl.ds(sub * RPT, RPT)])
        pltpu.sync_copy(cnt_sh.at[pl.ds(sub * RPT, RPT)],
                        cnt_out.at[core, pl.ds(sub * RPT, RPT)])

    return _emb_gather, _edge_agg


def _k1_body(aggP, cntP, h, alive, Wl, bl, Wr, p, hout_ref, score_ref):
    agg = aggP[0] + aggP[1]
    cnt = (cntP[0] + cntP[1])[:, 0:1]
    mean = agg / jnp.maximum(cnt, 1.0)
    pre = mean @ Wl[...].T + bl[...] + h[...] @ Wr[...].T
    ho = jnp.maximum(pre, 0.0)
    hout_ref[...] = ho
    pv = p[...]
    pn = jnp.sqrt(jnp.sum(pv * pv))
    s = jnp.tanh(jnp.sum(ho * pv, axis=1, keepdims=True) / pn)
    score_ref[...] = jnp.where(alive[...] > 0.0, s, NEG)


_k1 = pl.pallas_call(
    _k1_body,
    grid=(NP // BK1,),
    in_specs=[
        pl.BlockSpec((NC, BK1, D), lambda i: (0, i, 0)),
        pl.BlockSpec((NC, BK1, 16), lambda i: (0, i, 0)),
        pl.BlockSpec((BK1, D), lambda i: (i, 0)),
        pl.BlockSpec((BK1, 1), lambda i: (i, 0)),
        pl.BlockSpec((D, D), lambda i: (0, 0)),
        pl.BlockSpec((1, D), lambda i: (0, 0)),
        pl.BlockSpec((D, D), lambda i: (0, 0)),
        pl.BlockSpec((1, D), lambda i: (0, 0)),
    ],
    out_specs=[
        pl.BlockSpec((BK1, D), lambda i: (i, 0)),
        pl.BlockSpec((BK1, 1), lambda i: (i, 0)),
    ],
    out_shape=[
        jax.ShapeDtypeStruct((NP, D), jnp.float32),
        jax.ShapeDtypeStruct((NP, 1), jnp.float32),
    ],
)


def _k23_body(s_col, s_row, b_col, b_row, p_col, p_row, hout,
              hp_out, keep_out, prio_out, xs_out,
              kr_s, Kr_s, kc_s, st_s, cf_s, S_s, M_s):
    step = pl.program_id(0)

    @pl.when(step == 0)
    def _():
        sr = s_row[...]
        br = b_row[...]
        gi_col = lax.broadcasted_iota(jnp.int32, (G, 1), 0).astype(jnp.float32)
        alive_r = jnp.where(sr > -1.5, 1.0, 0.0)
        oh_gn = jnp.where(br == gi_col, 1.0, 0.0)
        i0 = lax.broadcasted_iota(jnp.int32, (G, G), 0)
        i1 = lax.broadcasted_iota(jnp.int32, (G, G), 1)
        eye = jnp.where(i0 == i1, 1.0, 0.0)
        ut = jnp.where(i0 < i1, 1.0, 0.0)
        ones_g = jnp.ones((1, G), jnp.float32)

        def tr(col):
            return ones_g @ (eye * col)

        kc = jnp.ceil(jnp.float32(0.8)
                      * jnp.sum(oh_gn * alive_r, axis=1, keepdims=True))
        kc_s[...] = kc
        kr = tr(kc)
        kr_s[...] = kr
        Kr_s[...] = kr @ ut
        cf = tr(jnp.sum(oh_gn, axis=1, keepdims=True))
        cf_s[...] = cf
        st_s[...] = cf @ ut
        S_s[...] = jnp.zeros((G, D), jnp.float32)
        M_s[...] = jnp.full((G, D), -jnp.inf, jnp.float32)

    @pl.when(step > 0)
    def _():
        i = step - 1
        si = s_col[...]
        bi = b_col[...]
        pi = p_col[...]
        gi_row = lax.broadcasted_iota(jnp.int32, (1, G), 1).astype(jnp.float32)
        gi_col = lax.broadcasted_iota(jnp.int32, (G, 1), 0).astype(jnp.float32)
        bmin = bi[0, 0]
        bmax = bi[BK - 1, 0]
        lo = jnp.sum(jnp.where(gi_row == bmin, st_s[...], 0.0))
        hi = jnp.sum(jnp.where(gi_row == bmax, st_s[...] + cf_s[...], 0.0))
        jlo = lo.astype(jnp.int32) // JC
        jhi = (hi.astype(jnp.int32) + JC - 1) // JC

        def jbody(jc, rank):
            sj = s_row[:, pl.ds(jc * JC, JC)]
            bj = b_row[:, pl.ds(jc * JC, JC)]
            pj = p_row[:, pl.ds(jc * JC, JC)]
            cmp = (bj == bi) & ((sj > si) | ((sj == si) & (pj < pi)))
            return rank + jnp.sum(jnp.where(cmp, 1.0, 0.0), axis=1,
                                  keepdims=True)

        rank = lax.fori_loop(jlo, jhi, jbody, jnp.zeros((BK, 1), jnp.float32))

        ohi = jnp.where(bi == gi_row, 1.0, 0.0)
        k_i = jnp.sum(ohi * kr_s[...], axis=1, keepdims=True)
        K_i = jnp.sum(ohi * Kr_s[...], axis=1, keepdims=True)
        keep = rank < k_i
        keepf = jnp.where(keep, 1.0, 0.0)
        gate = jnp.where(keep, si, 0.0)
        hp = hout[...] * gate
        hp_out[...] = hp
        keep_out[...] = keepf
        prio_out[...] = K_i + rank

        brow_i = b_row[:, pl.ds(i * BK, BK)]
        ohg = jnp.where(brow_i == gi_col, 1.0, 0.0)
        S_s[...] = S_s[...] + ohg @ hp

        for g in range(G):
            @pl.when((bmin <= jnp.float32(g)) & (jnp.float32(g) <= bmax))
            def _(g=g):
                msk = (bi == jnp.float32(g)) & keep
                mrow = jnp.max(jnp.where(msk, hp, -jnp.inf), axis=0,
                               keepdims=True)
                M_s[g:g + 1, :] = jnp.maximum(M_s[g:g + 1, :], mrow)

    @pl.when(step == NB)
    def _():
        mean = S_s[...] / jnp.maximum(kc_s[...], 1.0)
        mx = M_s[...]
        mx = jnp.where(mx == -jnp.inf, 0.0, mx)
        xs_out[:, 0:D] = mx
        xs_out[:, D:2 * D] = mean


def _bk_map(i):
    return (jnp.maximum(i - 1, 0), 0)


_k23 = pl.pallas_call(
    _k23_body,
    grid=(NB + 1,),
    in_specs=[
        pl.BlockSpec((BK, 1), _bk_map),
        pl.BlockSpec((1, NP), lambda i: (0, 0)),
        pl.BlockSpec((BK, 1), _bk_map),
        pl.BlockSpec((1, NP), lambda i: (0, 0)),
        pl.BlockSpec((BK, 1), _bk_map),
        pl.BlockSpec((1, NP), lambda i: (0, 0)),
        pl.BlockSpec((BK, D), _bk_map),
    ],
    out_specs=[
        pl.BlockSpec((BK, D), _bk_map),
        pl.BlockSpec((BK, 1), _bk_map),
        pl.BlockSpec((BK, 1), _bk_map),
        pl.BlockSpec((G, 2 * D), lambda i: (0, 0)),
    ],
    out_shape=[
        jax.ShapeDtypeStruct((NP, D), jnp.float32),
        jax.ShapeDtypeStruct((NP, 1), jnp.float32),
        jax.ShapeDtypeStruct((NP, 1), jnp.float32),
        jax.ShapeDtypeStruct((G, 2 * D), jnp.float32),
    ],
    scratch_shapes=[
        pltpu.VMEM((1, G), jnp.float32),
        pltpu.VMEM((1, G), jnp.float32),
        pltpu.VMEM((G, 1), jnp.float32),
        pltpu.VMEM((1, G), jnp.float32),
        pltpu.VMEM((1, G), jnp.float32),
        pltpu.VMEM((G, D), jnp.float32),
        pltpu.VMEM((G, D), jnp.float32),
    ],
)


def _k4_body(z1, z2, z3, W1, b1, W2, b2, W3, b3, o_ref):
    z = z1[...] + z2[...] + z3[...]
    h = jnp.maximum(z @ W1[...].T + b1[...], 0.0)
    h = jnp.maximum(h @ W2[...].T + b2[...], 0.0)
    s = jnp.sum(h * W3[...], axis=1, keepdims=True) + b3[...]
    o_ref[...] = jax.nn.sigmoid(s)


_k4 = pl.pallas_call(
    _k4_body,
    out_shape=jax.ShapeDtypeStruct((G, 1), jnp.float32),
)


def kernel(x, edge_index, batch, emb, Wl1, bl1, Wr1, p1, Wl2, bl2, Wr2, p2,
           Wl3, bl3, Wr3, p3, W1, b1, W2, b2, W3, b3):
    emb_gather, edge_agg = _sc_kernels()
    ids_p = jnp.concatenate(
        [x[:, 0].astype(jnp.int32), jnp.zeros((NP - N,), jnp.int32)])
    h = emb_gather(emb, ids_p)

    epad = jnp.full((EP - E,), PAD_NODE, jnp.int32)
    src_p = jnp.concatenate([edge_index[0].astype(jnp.int32), epad])
    dst_p = jnp.concatenate([edge_index[1].astype(jnp.int32), epad])

    batch_pf = jnp.concatenate(
        [batch.astype(jnp.float32), jnp.full((NP - N,), 63.0, jnp.float32)])
    b_col = batch_pf.reshape(NP, 1)
    b_row = batch_pf.reshape(1, NP)
    prio = jnp.arange(NP, dtype=jnp.float32)
    p_col = prio.reshape(NP, 1)
    p_row = prio.reshape(1, NP)
    alive_col = (jnp.arange(NP) < N).astype(jnp.float32).reshape(NP, 1)

    xs_list = []
    for (Wl, bl, Wr, p) in ((Wl1, bl1, Wr1, p1), (Wl2, bl2, Wr2, p2),
                            (Wl3, bl3, Wr3, p3)):
        alive16 = jnp.broadcast_to(alive_col, (NP, 16))
        aggP, cntP = edge_agg(h, alive16, src_p, dst_p)
        hout, s_col = _k1(aggP, cntP, h, alive_col,
                          Wl, bl.reshape(1, D), Wr, p.reshape(1, D))
        s_row = s_col.reshape(1, NP)
        h, keep_col, prio_col, xs = _k23(s_col, s_row, b_col, b_row,
                                         p_col, p_row, hout)
        xs_list.append(xs)
        alive_col = keep_col
        p_col = prio_col
        p_row = prio_col.reshape(1, NP)

    out = _k4(xs_list[0], xs_list[1], xs_list[2],
              W1, b1.reshape(1, -1), W2, b2.reshape(1, -1),
              W3, b3.reshape(1, 1))
    return out[:, 0]


# speedup vs baseline: 1.1559x; 1.0085x over previous
import functools

import jax
import jax.numpy as jnp
from jax import lax
from jax.experimental import pallas as pl
from jax.experimental.pallas import tpu as pltpu
from jax.experimental.pallas import tpu_sc as plsc

N = 10000
NP = 10240
E = 320000
EP = 327680
G = 64
D = 128
NC = 2
NS = 16
NW = NC * NS
BPW = NP // NW
EPW = EP // NW
EC = 64
NCHUNK = EPW // EC
RPT = NP // NS
PAD_NODE = 10200

BK1 = 1024
BK = 512
NB = NP // BK
JC = 1024
NEG = -2.0
DT = 144

@functools.lru_cache(maxsize=None)
def _sc_kernels():
    mesh = plsc.VectorSubcoreMesh(core_axis_name="c", subcore_axis_name="s",
                                  num_cores=NC, num_subcores=NS)

    @functools.partial(
        pl.kernel,
        out_type=jax.ShapeDtypeStruct((NP, D), jnp.float32),
        mesh=mesh,
        scratch_types=[
            pltpu.VMEM((BPW,), jnp.int32),
            pltpu.VMEM((BPW, D), jnp.float32),
            pltpu.SemaphoreType.DMA,
        ],
    )
    def _emb_gather(table_hbm, idx_hbm, out_hbm, idx_v, rows_v, sem):
        wid = lax.axis_index("s") * NC + lax.axis_index("c")
        base = wid * BPW
        pltpu.sync_copy(idx_hbm.at[pl.ds(base, BPW)], idx_v)
        for off, sz in ((0, 128), (128, 128), (256, 64)):
            pltpu.async_copy(
                table_hbm.at[idx_v.at[pl.ds(off, sz)]],
                rows_v.at[pl.ds(off, sz)],
                sem,
            ).wait()
        pltpu.sync_copy(rows_v, out_hbm.at[pl.ds(base, BPW)])

    @functools.partial(
        pl.kernel,
        out_type=[
            jax.ShapeDtypeStruct((NC, NP, D), jnp.float32),
            jax.ShapeDtypeStruct((NC, NP, 16), jnp.float32),
        ],
        mesh=mesh,
        scratch_types=(
            [pltpu.VMEM((EC, D), jnp.float32)] * 4
            + [pltpu.VMEM((EC, 16), jnp.float32)] * 4
            + [pltpu.VMEM((EC,), jnp.int32)] * 16
            + [
                pltpu.VMEM_SHARED((NP, D), jnp.float32),
                pltpu.VMEM_SHARED((NP, 16), jnp.float32),
            ]
            + [pltpu.SemaphoreType.DMA] * 16
        ),
        compiler_params=pltpu.CompilerParams(use_tc_tiling_on_sc=False),
    )
    def _edge_agg(h_hbm, alive16_hbm, src_hbm, dst_hbm, agg_out, cnt_out,
                  rows_0, rows_1, rows_2, rows_3,
                  arows_0, arows_1, arows_2, arows_3,
                  sidx_0, sidx_1, sidx_2, sidx_3,
                  sidx_4, sidx_5, sidx_6, sidx_7,
                  didx_0, didx_1, didx_2, didx_3,
                  didx_4, didx_5, didx_6, didx_7,
                  acc_sh, cnt_sh,
                  sem_gr0, sem_gr1, sem_gr2, sem_gr3,
                  sem_ga0, sem_ga1, sem_ga2, sem_ga3,
                  sem_i0, sem_i1, sem_i2, sem_i3,
                  sem_i4, sem_i5, sem_i6, sem_i7):
        rows = (rows_0, rows_1, rows_2, rows_3)
        arows = (arows_0, arows_1, arows_2, arows_3)
        sidx = (sidx_0, sidx_1, sidx_2, sidx_3,
                sidx_4, sidx_5, sidx_6, sidx_7)
        didx = (didx_0, didx_1, didx_2, didx_3,
                didx_4, didx_5, didx_6, didx_7)
        sem_gr = (sem_gr0, sem_gr1, sem_gr2, sem_gr3)
        sem_ga = (sem_ga0, sem_ga1, sem_ga2, sem_ga3)
        sem_i = (sem_i0, sem_i1, sem_i2, sem_i3,
                 sem_i4, sem_i5, sem_i6, sem_i7)

        core = lax.axis_index("c")
        sub = lax.axis_index("s")
        wid = sub * NC + core
        ebase = wid * EPW

        zv = jnp.zeros((16,), jnp.float32)

        def zrow(i, c):
            rows_0[i // 8, pl.ds((i % 8) * 16, 16)] = zv
            return c

        lax.fori_loop(0, EC * 8, zrow, 0)

        def zarow(i, c):
            arows_0[i, pl.ds(0, 16)] = zv
            return c

        lax.fori_loop(0, EC, zarow, 0)

        for r in range(RPT // EC):
            pltpu.sync_copy(rows_0,
                            acc_sh.at[pl.ds(sub * RPT + r * EC, EC)])
            pltpu.sync_copy(arows_0,
                            cnt_sh.at[pl.ds(sub * RPT + r * EC, EC)])
        plsc.subcore_barrier()

        def issue_idx(c, q):
            pltpu.async_copy(src_hbm.at[pl.ds(ebase + c * EC, EC)],
                             sidx[q], sem_i[q])
            pltpu.async_copy(dst_hbm.at[pl.ds(ebase + c * EC, EC)],
                             didx[q], sem_i[q])

        def wait_idx(q):
            pltpu.make_async_copy(src_hbm.at[pl.ds(0, EC)], sidx[q],
                                  sem_i[q]).wait()
            pltpu.make_async_copy(dst_hbm.at[pl.ds(0, EC)], didx[q],
                                  sem_i[q]).wait()

        def issue_g(q, b):
            pltpu.async_copy(h_hbm.at[sidx[q]], rows[b], sem_gr[b])
            pltpu.async_copy(alive16_hbm.at[sidx[q]], arows[b], sem_ga[b])

        def wait_g(b):
            pltpu.make_async_copy(h_hbm.at[sidx[0]], rows[b],
                                  sem_gr[b]).wait()
            pltpu.make_async_copy(alive16_hbm.at[sidx[0]], arows[b],
                                  sem_ga[b]).wait()

        def sync_s(q, b):
            pltpu.sync_copy(rows[b], acc_sh.at[didx[q]], add=True)
            pltpu.sync_copy(arows[b], cnt_sh.at[didx[q]], add=True)

        for q in range(8):
            issue_idx(q, q)
        for b in range(4):
            wait_idx(b)
            issue_g(b, b)

        def ring(g, carry):
            for bb in range(8):
                c = g * 8 + bb
                b = bb % 4
                wait_g(b)
                sync_s(bb, b)

                @pl.when(c + 4 < NCHUNK)
                def _(c=c, bb=bb, b=b):
                    wait_idx((bb + 4) % 8)
                    issue_g((bb + 4) % 8, b)

                @pl.when(c + 8 < NCHUNK)
                def _(c=c, bb=bb):
                    issue_idx(c + 8, bb)
            return carry

        lax.fori_loop(0, NCHUNK // 8, ring, 0)

        plsc.subcore_barrier()
        pltpu.sync_copy(acc_sh.at[pl.ds(sub * RPT, RPT)],
                        agg_out.at[core, pl.ds(sub * RPT, RPT)])
        pltpu.sync_copy(cnt_sh.at[pl.ds(sub * RPT, RPT)],
                        cnt_out.at[core, pl.ds(sub * RPT, RPT)])

    return _emb_gather, _edge_agg


def _k1_body(aggP, cntP, h, alive, Wl, bl, Wr, p, hout_ref, score_ref):
    agg = aggP[0] + aggP[1]
    cnt = (cntP[0] + cntP[1])[:, 0:1]
    mean = agg / jnp.maximum(cnt, 1.0)
    pre = mean @ Wl[...].T + bl[...] + h[...] @ Wr[...].T
    ho = jnp.maximum(pre, 0.0)
    hout_ref[...] = ho
    pv = p[...]
    pn = jnp.sqrt(jnp.sum(pv * pv))
    s = jnp.tanh(jnp.sum(ho * pv, axis=1, keepdims=True) / pn)
    score_ref[...] = jnp.where(alive[...] > 0.0, s, NEG)


_k1 = pl.pallas_call(
    _k1_body,
    grid=(NP // BK1,),
    in_specs=[
        pl.BlockSpec((NC, BK1, D), lambda i: (0, i, 0)),
        pl.BlockSpec((NC, BK1, 16), lambda i: (0, i, 0)),
        pl.BlockSpec((BK1, D), lambda i: (i, 0)),
        pl.BlockSpec((BK1, 1), lambda i: (i, 0)),
        pl.BlockSpec((D, D), lambda i: (0, 0)),
        pl.BlockSpec((1, D), lambda i: (0, 0)),
        pl.BlockSpec((D, D), lambda i: (0, 0)),
        pl.BlockSpec((1, D), lambda i: (0, 0)),
    ],
    out_specs=[
        pl.BlockSpec((BK1, D), lambda i: (i, 0)),
        pl.BlockSpec((BK1, 1), lambda i: (i, 0)),
    ],
    out_shape=[
        jax.ShapeDtypeStruct((NP, D), jnp.float32),
        jax.ShapeDtypeStruct((NP, 1), jnp.float32),
    ],
)


def _k23_body(s_col, s_row, b_col, b_row, p_col, p_row, hout,
              hp_out, keep_out, prio_out, xs_out,
              kr_s, Kr_s, kc_s, st_s, cf_s, S_s, M_s):
    step = pl.program_id(0)

    @pl.when(step == 0)
    def _():
        sr = s_row[...]
        br = b_row[...]
        gi_col = lax.broadcasted_iota(jnp.int32, (G, 1), 0).astype(jnp.float32)
        alive_r = jnp.where(sr > -1.5, 1.0, 0.0)
        oh_gn = jnp.where(br == gi_col, 1.0, 0.0)
        i0 = lax.broadcasted_iota(jnp.int32, (G, G), 0)
        i1 = lax.broadcasted_iota(jnp.int32, (G, G), 1)
        eye = jnp.where(i0 == i1, 1.0, 0.0)
        ut = jnp.where(i0 < i1, 1.0, 0.0)
        ones_g = jnp.ones((1, G), jnp.float32)

        def tr(col):
            return ones_g @ (eye * col)

        kc = jnp.ceil(jnp.float32(0.8)
                      * jnp.sum(oh_gn * alive_r, axis=1, keepdims=True))
        kc_s[...] = kc
        kr = tr(kc)
        kr_s[...] = kr
        Kr_s[...] = kr @ ut
        cf = tr(jnp.sum(oh_gn, axis=1, keepdims=True))
        cf_s[...] = cf
        st_s[...] = cf @ ut
        S_s[...] = jnp.zeros((G, D), jnp.float32)
        M_s[...] = jnp.full((G, D), -jnp.inf, jnp.float32)

    @pl.when(step > 0)
    def _():
        i = step - 1
        si = s_col[...]
        bi = b_col[...]
        pi = p_col[...]
        gi_row = lax.broadcasted_iota(jnp.int32, (1, G), 1).astype(jnp.float32)
        gi_col = lax.broadcasted_iota(jnp.int32, (G, 1), 0).astype(jnp.float32)
        bmin = bi[0, 0]
        bmax = bi[BK - 1, 0]
        lo = jnp.sum(jnp.where(gi_row == bmin, st_s[...], 0.0))
        hi = jnp.sum(jnp.where(gi_row == bmax, st_s[...] + cf_s[...], 0.0))
        jlo = lo.astype(jnp.int32) // JC
        jhi = (hi.astype(jnp.int32) + JC - 1) // JC

        def jbody(jc, rank):
            sj = s_row[:, pl.ds(jc * JC, JC)]
            bj = b_row[:, pl.ds(jc * JC, JC)]
            pj = p_row[:, pl.ds(jc * JC, JC)]
            cmp = (bj == bi) & ((sj > si) | ((sj == si) & (pj < pi)))
            return rank + jnp.sum(jnp.where(cmp, 1.0, 0.0), axis=1,
                                  keepdims=True)

        rank = lax.fori_loop(jlo, jhi, jbody, jnp.zeros((BK, 1), jnp.float32))

        ohi = jnp.where(bi == gi_row, 1.0, 0.0)
        k_i = jnp.sum(ohi * kr_s[...], axis=1, keepdims=True)
        K_i = jnp.sum(ohi * Kr_s[...], axis=1, keepdims=True)
        keep = rank < k_i
        keepf = jnp.where(keep, 1.0, 0.0)
        gate = jnp.where(keep, si, 0.0)
        hp = hout[...] * gate
        hp_out[...] = hp
        keep_out[...] = keepf
        prio_out[...] = K_i + rank

        brow_i = b_row[:, pl.ds(i * BK, BK)]
        ohg = jnp.where(brow_i == gi_col, 1.0, 0.0)
        S_s[...] = S_s[...] + ohg @ hp

        for g in range(G):
            @pl.when((bmin <= jnp.float32(g)) & (jnp.float32(g) <= bmax))
            def _(g=g):
                msk = (bi == jnp.float32(g)) & keep
                mrow = jnp.max(jnp.where(msk, hp, -jnp.inf), axis=0,
                               keepdims=True)
                M_s[g:g + 1, :] = jnp.maximum(M_s[g:g + 1, :], mrow)

    @pl.when(step == NB)
    def _():
        mean = S_s[...] / jnp.maximum(kc_s[...], 1.0)
        mx = M_s[...]
        mx = jnp.where(mx == -jnp.inf, 0.0, mx)
        xs_out[:, 0:D] = mx
        xs_out[:, D:2 * D] = mean


def _bk_map(i):
    return (jnp.maximum(i - 1, 0), 0)


_k23 = pl.pallas_call(
    _k23_body,
    grid=(NB + 1,),
    in_specs=[
        pl.BlockSpec((BK, 1), _bk_map),
        pl.BlockSpec((1, NP), lambda i: (0, 0)),
        pl.BlockSpec((BK, 1), _bk_map),
        pl.BlockSpec((1, NP), lambda i: (0, 0)),
        pl.BlockSpec((BK, 1), _bk_map),
        pl.BlockSpec((1, NP), lambda i: (0, 0)),
        pl.BlockSpec((BK, D), _bk_map),
    ],
    out_specs=[
        pl.BlockSpec((BK, D), _bk_map),
        pl.BlockSpec((BK, 1), _bk_map),
        pl.BlockSpec((BK, 1), _bk_map),
        pl.BlockSpec((G, 2 * D), lambda i: (0, 0)),
    ],
    out_shape=[
        jax.ShapeDtypeStruct((NP, D), jnp.float32),
        jax.ShapeDtypeStruct((NP, 1), jnp.float32),
        jax.ShapeDtypeStruct((NP, 1), jnp.float32),
        jax.ShapeDtypeStruct((G, 2 * D), jnp.float32),
    ],
    scratch_shapes=[
        pltpu.VMEM((1, G), jnp.float32),
        pltpu.VMEM((1, G), jnp.float32),
        pltpu.VMEM((G, 1), jnp.float32),
        pltpu.VMEM((1, G), jnp.float32),
        pltpu.VMEM((1, G), jnp.float32),
        pltpu.VMEM((G, D), jnp.float32),
        pltpu.VMEM((G, D), jnp.float32),
    ],
)


def _k4_body(z1, z2, z3, W1, b1, W2, b2, W3, b3, o_ref):
    z = z1[...] + z2[...] + z3[...]
    h = jnp.maximum(z @ W1[...].T + b1[...], 0.0)
    h = jnp.maximum(h @ W2[...].T + b2[...], 0.0)
    s = jnp.sum(h * W3[...], axis=1, keepdims=True) + b3[...]
    o_ref[...] = jax.nn.sigmoid(s)


_k4 = pl.pallas_call(
    _k4_body,
    out_shape=jax.ShapeDtypeStruct((G, 1), jnp.float32),
)


def kernel(x, edge_index, batch, emb, Wl1, bl1, Wr1, p1, Wl2, bl2, Wr2, p2,
           Wl3, bl3, Wr3, p3, W1, b1, W2, b2, W3, b3):
    emb_gather, edge_agg = _sc_kernels()
    ids_p = jnp.concatenate(
        [x[:, 0].astype(jnp.int32), jnp.zeros((NP - N,), jnp.int32)])
    h = emb_gather(emb, ids_p)

    epad = jnp.full((EP - E,), PAD_NODE, jnp.int32)
    src_p = jnp.concatenate([edge_index[0].astype(jnp.int32), epad])
    dst_p = jnp.concatenate([edge_index[1].astype(jnp.int32), epad])

    batch_pf = jnp.concatenate(
        [batch.astype(jnp.float32), jnp.full((NP - N,), 63.0, jnp.float32)])
    b_col = batch_pf.reshape(NP, 1)
    b_row = batch_pf.reshape(1, NP)
    prio = jnp.arange(NP, dtype=jnp.float32)
    p_col = prio.reshape(NP, 1)
    p_row = prio.reshape(1, NP)
    alive_col = (jnp.arange(NP) < N).astype(jnp.float32).reshape(NP, 1)

    xs_list = []
    for (Wl, bl, Wr, p) in ((Wl1, bl1, Wr1, p1), (Wl2, bl2, Wr2, p2),
                            (Wl3, bl3, Wr3, p3)):
        alive16 = jnp.broadcast_to(alive_col, (NP, 16))
        aggP, cntP = edge_agg(h, alive16, src_p, dst_p)
        hout, s_col = _k1(aggP, cntP, h, alive_col,
                          Wl, bl.reshape(1, D), Wr, p.reshape(1, D))
        s_row = s_col.reshape(1, NP)
        h, keep_col, prio_col, xs = _k23(s_col, s_row, b_col, b_row,
                                         p_col, p_row, hout)
        xs_list.append(xs)
        alive_col = keep_col
        p_col = prio_col
        p_row = prio_col.reshape(1, NP)

    out = _k4(xs_list[0], xs_list[1], xs_list[2],
              W1, b1.reshape(1, -1), W2, b2.reshape(1, -1),
              W3, b3.reshape(1, 1))
    return out[:, 0]
